# baseline (speedup 1.0000x reference)
import functools

import jax
import jax.numpy as jnp
import numpy as np
from jax import lax
from jax.experimental import pallas as pl
from jax.experimental.pallas import tpu as pltpu
from jax.experimental.pallas import tpu_sc as plsc

_N = 10000
_E = 320000
_D = 128
_NG = 64
_NQ = 8
_DIM = 256
_QL = 2
_NCLS = 10

_NC, _NS = 2, 16
_NW = _NC * _NS
_CHUNK = 128
_SUP = 8
_NSUP = 10
_CPT = _SUP * _NSUP
_EPAD = _NW * _CPT * _CHUNK
_CPT0, _CPT1 = 80, 80
_NSUP0, _NSUP1 = _CPT0 // _SUP, _CPT1 // _SUP
_NPAD = 10112
_RPT = _NPAD // _NS
_DEGW = 16


def _build_quantum_mats():
    X = np.zeros((_NQ, _DIM, _DIM), np.float32)
    M = np.zeros((_NQ, _DIM), np.float32)
    C = np.zeros((_NQ, _DIM, _DIM), np.float32)
    idx = np.arange(_DIM)
    for q in range(_NQ):
        bit = 1 << (_NQ - 1 - q)
        X[q, idx ^ bit, idx] = 1.0
        M[q] = np.where(idx & bit, 1.0, -1.0)
        ctrl_bit = (idx >> (_NQ - 1 - q)) & 1
        tgt = (q + 1) % _NQ
        perm = np.where(ctrl_bit == 1, idx ^ (1 << (_NQ - 1 - tgt)), idx)
        C[q, perm, idx] = 1.0
    CC = C[0]
    for q in range(1, _NQ):
        CC = CC @ C[q]
    S = np.zeros((_NQ, _DIM), np.float32)
    for q in range(_NQ):
        S[q] = 1.0 - 2.0 * ((idx >> (_NQ - 1 - q)) & 1)
    return X, M, np.ascontiguousarray(CC), np.ascontiguousarray(S.T)


_XM, _MM, _CCM, _SIGNS_T = _build_quantum_mats()


def _sc_degree(dst_pad):
    mesh = plsc.VectorSubcoreMesh(core_axis_name="c", subcore_axis_name="s")

    @functools.partial(
        pl.kernel,
        out_type=jax.ShapeDtypeStruct((_NW * _NPAD,), jnp.float32),
        mesh=mesh,
        scratch_types=[
            pltpu.VMEM((_SUP, _CHUNK), jnp.int32),
            pltpu.VMEM((_NPAD,), jnp.float32),
        ],
        compiler_params=pltpu.CompilerParams(needs_layout_passes=False),
    )
    def deg_kernel(dst_hbm, out_hbm, idx_v, hist):
        c = lax.axis_index("c")
        s = lax.axis_index("s")
        w = c * _NS + s

        def fill_zero(i, carry):
            hist[pl.ds(i * 16, 16)] = jnp.zeros((16,), jnp.float32)
            return carry

        lax.fori_loop(0, _NPAD // 16, fill_zero, 0)

        def sup(t, carry):
            row = w * _CPT + t * _SUP
            pltpu.sync_copy(dst_hbm.at[pl.ds(row, _SUP)], idx_v)
            for k in range(_SUP):
                for j in range(_CHUNK // 16):
                    iv = idx_v[k, pl.ds(j * 16, 16)]
                    cnt, last = plsc.scan_count(iv)
                    plsc.addupdate_scatter(hist, [iv],
                                           cnt.astype(jnp.float32), mask=last)
            return carry

        lax.fori_loop(0, _NSUP, sup, 0)

        pltpu.sync_copy(hist, out_hbm.at[pl.ds(w * _NPAD, _NPAD)])

    return deg_kernel(dst_pad)


def _sc_edge_scatter(hp, src_pad, dst_pad):
    mesh = plsc.VectorSubcoreMesh(core_axis_name="c", subcore_axis_name="s")

    @functools.partial(
        pl.kernel,
        out_type=jax.ShapeDtypeStruct((_NC * _NPAD, _D), jnp.float32),
        mesh=mesh,
        scratch_types=[
            pltpu.VMEM((_SUP, _CHUNK), jnp.int32),
            pltpu.VMEM((_SUP, _CHUNK), jnp.int32),
            pltpu.VMEM((_CHUNK, _D), jnp.float32),
            pltpu.VMEM((_CHUNK, _D), jnp.float32),
            pltpu.SemaphoreType.DMA,
            pltpu.SemaphoreType.DMA,
            pltpu.VMEM_SHARED((_NPAD, _D), jnp.float32),
        ],
    )
    def edge_kernel(hp_hbm, src_hbm, dst_hbm, out_hbm,
                    idx_v, didx_v, rows_a, rows_b,
                    sem_a, sem_b, acc_sh):
        c = lax.axis_index("c")
        s = lax.axis_index("s")
        nsup = jnp.where(c == 0, _NSUP0, _NSUP1)
        chunk_base = jnp.where(c == 0, s * _CPT0, _NS * _CPT0 + s * _CPT1)

        with jax.named_scope("acc_zero_init"):
            def fill_zero(i, carry):
                for k in range(_D // 16):
                    rows_a[i, pl.ds(k * 16, 16)] = jnp.zeros((16,),
                                                             jnp.float32)
                return carry

            lax.fori_loop(0, _CHUNK, fill_zero, 0)
            for k in range(_RPT // _CHUNK):
                pltpu.sync_copy(rows_a,
                                acc_sh.at[pl.ds(s * _RPT + k * _CHUNK,
                                                _CHUNK)])
            _REM = _RPT % _CHUNK
            pltpu.sync_copy(
                rows_a.at[pl.ds(0, _REM)],
                acc_sh.at[pl.ds(s * _RPT + (_RPT // _CHUNK) * _CHUNK, _REM)])
        plsc.subcore_barrier()

        rows = (rows_a, rows_b)
        sems = (sem_a, sem_b)

        def sup(t, carry):
            row = chunk_base + t * _SUP
            pltpu.sync_copy(src_hbm.at[pl.ds(row, _SUP)], idx_v)
            pltpu.sync_copy(dst_hbm.at[pl.ds(row, _SUP)], didx_v)
            cp = pltpu.async_copy(hp_hbm.at[idx_v.at[0]], rows[0], sems[0])
            for k in range(_SUP):
                nxt = None
                if k + 1 < _SUP:
                    nxt = pltpu.async_copy(hp_hbm.at[idx_v.at[k + 1]],
                                           rows[(k + 1) % 2],
                                           sems[(k + 1) % 2])
                cp.wait()
                pltpu.sync_copy(rows[k % 2], acc_sh.at[didx_v.at[k]],
                                add=True)
                cp = nxt
            return carry

        with jax.named_scope("edge_main"):
            lax.fori_loop(0, nsup, sup, 0)

        plsc.subcore_barrier()
        with jax.named_scope("acc_copy_out"):
            pltpu.sync_copy(
                acc_sh.at[pl.ds(s * _RPT, _RPT)],
                out_hbm.at[pl.ds(c * _NPAD + s * _RPT, _RPT)],
            )

    return edge_kernel(hp, src_pad, dst_pad)


def _tc_prescale(x, wt, deg_hists):
    def body(x_ref, wt_ref, dh_ref, hp_ref, dinv_ref):
        hists = dh_ref[...]
        ones_w = jnp.ones((_NW, 1), jnp.float32)
        deg = lax.dot_general(hists, ones_w, (((0,), (0,)), ((), ())),
                              preferred_element_type=jnp.float32,
                  precision=lax.Precision.HIGHEST) + 1.0
        dinv = lax.rsqrt(jnp.maximum(deg, 1e-12))
        dinv_ref[...] = jnp.broadcast_to(dinv, (_NPAD, _DEGW))
        h = jnp.dot(x_ref[...], wt_ref[...], preferred_element_type=jnp.float32,
                  precision=lax.Precision.HIGHEST)
        hp_ref[...] = h * dinv[:_N]

    return pl.pallas_call(
        body,
        out_shape=(
            jax.ShapeDtypeStruct((_N, _D), jnp.float32),
            jax.ShapeDtypeStruct((_NPAD, _DEGW), jnp.float32),
        ),
    )(x, wt, deg_hists)


def _tc_tail(accp, hp, dinv16, batch2, bg, wa_t, ba, qwf,
             xm, mm, cm, signs_t, wc1, wc2, bc):
    def body(accp_ref, hp_ref, dinv_ref, batch_ref, bg_ref, wa_ref, ba_ref,
             qwf_ref, xm_ref, mm_ref, cm_ref, st_ref, wc1_ref, wc2_ref,
             bc_ref, out_ref):
        av = accp_ref[...]
        a = av[:_N] + av[_NPAD:_NPAD + _N]
        dinv = dinv_ref[...][:_N, 0:1]
        g = dinv * (a + hp_ref[...]) + bg_ref[...]
        hr = jnp.maximum(g, 0.0)

        bv = batch_ref[...]
        gi = lax.broadcasted_iota(jnp.int32, (_NG, _N), 0)
        msk = (gi == bv).astype(jnp.float32)
        sums = jnp.dot(msk, hr, preferred_element_type=jnp.float32,
                  precision=lax.Precision.HIGHEST)
        counts = jnp.sum(msk, axis=1, keepdims=True)
        pooled = sums / jnp.maximum(counts, 1.0)

        thetas = (jnp.dot(pooled, wa_ref[...], preferred_element_type=jnp.float32,
                  precision=lax.Precision.HIGHEST)
                  + ba_ref[...] + qwf_ref[...])
        col = lax.broadcasted_iota(jnp.int32, (_NG, _DIM), 1)
        state = jnp.where(col == 0, 1.0, 0.0).astype(jnp.float32)
        mmv = mm_ref[...]
        for l in range(_QL):
            for q in range(_NQ):
                th = thetas[:, l * _NQ + q:l * _NQ + q + 1] * 0.5
                cth = jnp.cos(th)
                sth = jnp.sin(th)
                partner = jnp.dot(state, xm_ref[q],
                                  preferred_element_type=jnp.float32,
                  precision=lax.Precision.HIGHEST)
                state = cth * state + sth * (mmv[q:q + 1] * partner)
            state = jnp.dot(state, cm_ref[...],
                            preferred_element_type=jnp.float32,
                  precision=lax.Precision.HIGHEST)

        qout = jnp.dot(state * state, st_ref[...],
                       preferred_element_type=jnp.float32,
                  precision=lax.Precision.HIGHEST)
        out_ref[...] = (jnp.dot(pooled, wc1_ref[...],
                                preferred_element_type=jnp.float32,
                  precision=lax.Precision.HIGHEST)
                        + jnp.dot(qout, wc2_ref[...],
                                  preferred_element_type=jnp.float32,
                  precision=lax.Precision.HIGHEST)
                        + bc_ref[...])

    return pl.pallas_call(
        body,
        out_shape=jax.ShapeDtypeStruct((_NG, _NCLS), jnp.float32),
    )(accp, hp, dinv16, batch2, bg, wa_t, ba, qwf, xm, mm, cm, signs_t,
      wc1, wc2, bc)


def kernel(x, edge_index, batch, W_gcn, b_gcn, W_ang, b_ang, q_weights,
           W_cls, b_cls):
    src = edge_index[0].astype(jnp.int32)
    dst = edge_index[1].astype(jnp.int32)
    pad = _EPAD - _E
    src_pad = jnp.concatenate(
        [src, jnp.zeros((pad,), jnp.int32)]).reshape(_NW * _CPT, _CHUNK)
    pad_dst = _N + (jnp.arange(pad, dtype=jnp.int32) % (_NPAD - _N))
    dst_pad = jnp.concatenate([dst, pad_dst]).reshape(_NW * _CPT, _CHUNK)

    deg_hists = _sc_degree(dst_pad).reshape(_NW, _NPAD)
    hp, dinv16 = _tc_prescale(x, W_gcn.T, deg_hists)
    accp = _sc_edge_scatter(hp, src_pad, dst_pad)

    out = _tc_tail(
        accp, hp, dinv16,
        batch.astype(jnp.int32).reshape(1, _N),
        b_gcn.reshape(1, _D), W_ang.T, b_ang.reshape(1, -1),
        q_weights.reshape(1, -1),
        jnp.asarray(_XM), jnp.asarray(_MM), jnp.asarray(_CCM),
        jnp.asarray(_SIGNS_T),
        W_cls[:, :_D].T, W_cls[:, _D:].T, b_cls.reshape(1, -1),
    )
    return out


# speedup vs baseline: 2.2342x; 2.2342x over previous
import functools

import jax
import jax.numpy as jnp
import numpy as np
from jax import lax
from jax.experimental import pallas as pl
from jax.experimental.pallas import tpu as pltpu
from jax.experimental.pallas import tpu_sc as plsc

_N = 10000
_E = 320000
_D = 128
_NG = 64
_NQ = 8
_DIM = 256
_QL = 2
_NCLS = 10

_NC, _NS = 2, 16
_NW = _NC * _NS
_CHUNK = 128
_SUP = 8
_NSUP = 10
_CPT = _SUP * _NSUP
_EPAD = _NW * _CPT * _CHUNK
_CPT0, _CPT1 = 80, 80
_NSUP0, _NSUP1 = _CPT0 // _SUP, _CPT1 // _SUP
_NPAD = 10112
_RPT = _NPAD // _NS
_DEGW = 16


def _build_quantum_mats():
    X = np.zeros((_NQ, _DIM, _DIM), np.float32)
    M = np.zeros((_NQ, _DIM), np.float32)
    C = np.zeros((_NQ, _DIM, _DIM), np.float32)
    idx = np.arange(_DIM)
    for q in range(_NQ):
        bit = 1 << (_NQ - 1 - q)
        X[q, idx ^ bit, idx] = 1.0
        M[q] = np.where(idx & bit, 1.0, -1.0)
        ctrl_bit = (idx >> (_NQ - 1 - q)) & 1
        tgt = (q + 1) % _NQ
        perm = np.where(ctrl_bit == 1, idx ^ (1 << (_NQ - 1 - tgt)), idx)
        C[q, perm, idx] = 1.0
    CC = C[0]
    for q in range(1, _NQ):
        CC = CC @ C[q]
    S = np.zeros((_NQ, _DIM), np.float32)
    for q in range(_NQ):
        S[q] = 1.0 - 2.0 * ((idx >> (_NQ - 1 - q)) & 1)
    return X, M, np.ascontiguousarray(CC), np.ascontiguousarray(S.T)


_XM, _MM, _CCM, _SIGNS_T = _build_quantum_mats()


def _sc_degree(dst_pad):
    mesh = plsc.VectorSubcoreMesh(core_axis_name="c", subcore_axis_name="s")

    @functools.partial(
        pl.kernel,
        out_type=jax.ShapeDtypeStruct((_NW * _NPAD,), jnp.float32),
        mesh=mesh,
        scratch_types=[
            pltpu.VMEM((_SUP, _CHUNK), jnp.int32),
            pltpu.VMEM((_NPAD,), jnp.float32),
        ],
        compiler_params=pltpu.CompilerParams(needs_layout_passes=False),
    )
    def deg_kernel(dst_hbm, out_hbm, idx_v, hist):
        c = lax.axis_index("c")
        s = lax.axis_index("s")
        w = c * _NS + s

        def fill_zero(i, carry):
            hist[pl.ds(i * 16, 16)] = jnp.zeros((16,), jnp.float32)
            return carry

        lax.fori_loop(0, _NPAD // 16, fill_zero, 0)

        def sup(t, carry):
            row = w * _CPT + t * _SUP
            pltpu.sync_copy(dst_hbm.at[pl.ds(row, _SUP)], idx_v)
            for k in range(_SUP):
                for j in range(_CHUNK // 16):
                    iv = idx_v[k, pl.ds(j * 16, 16)]
                    cnt, last = plsc.scan_count(iv)
                    plsc.addupdate_scatter(hist, [iv],
                                           cnt.astype(jnp.float32), mask=last)
            return carry

        lax.fori_loop(0, _NSUP, sup, 0)

        pltpu.sync_copy(hist, out_hbm.at[pl.ds(w * _NPAD, _NPAD)])

    return deg_kernel(dst_pad)


def _sc_edge_scatter(hp, src_pad, dst_pad):
    mesh = plsc.VectorSubcoreMesh(core_axis_name="c", subcore_axis_name="s")

    @functools.partial(
        pl.kernel,
        out_type=jax.ShapeDtypeStruct((_NC * _NPAD, _D), jnp.float32),
        mesh=mesh,
        scratch_types=[
            pltpu.VMEM((_SUP, _CHUNK), jnp.int32),
            pltpu.VMEM((_SUP, _CHUNK), jnp.int32),
            pltpu.VMEM((_CHUNK, _D), jnp.float32),
            pltpu.VMEM((_CHUNK, _D), jnp.float32),
            pltpu.SemaphoreType.DMA,
            pltpu.SemaphoreType.DMA,
            pltpu.VMEM_SHARED((_NPAD, _D), jnp.float32),
        ],
    )
    def edge_kernel(hp_hbm, src_hbm, dst_hbm, out_hbm,
                    idx_v, didx_v, rows_a, rows_b,
                    sem_a, sem_b, acc_sh):
        c = lax.axis_index("c")
        s = lax.axis_index("s")
        nsup = jnp.where(c == 0, _NSUP0, _NSUP1)
        chunk_base = jnp.where(c == 0, s * _CPT0, _NS * _CPT0 + s * _CPT1)

        with jax.named_scope("acc_zero_init"):
            def fill_zero(i, carry):
                for k in range(_D // 16):
                    rows_a[i, pl.ds(k * 16, 16)] = jnp.zeros((16,),
                                                             jnp.float32)
                return carry

            lax.fori_loop(0, _CHUNK, fill_zero, 0)
            for k in range(_RPT // _CHUNK):
                pltpu.sync_copy(rows_a,
                                acc_sh.at[pl.ds(s * _RPT + k * _CHUNK,
                                                _CHUNK)])
            _REM = _RPT % _CHUNK
            pltpu.sync_copy(
                rows_a.at[pl.ds(0, _REM)],
                acc_sh.at[pl.ds(s * _RPT + (_RPT // _CHUNK) * _CHUNK, _REM)])
        plsc.subcore_barrier()

        rows = (rows_a, rows_b)
        sems = (sem_a, sem_b)

        def sup(t, carry):
            row = chunk_base + t * _SUP
            pltpu.sync_copy(src_hbm.at[pl.ds(row, _SUP)], idx_v)
            pltpu.sync_copy(dst_hbm.at[pl.ds(row, _SUP)], didx_v)
            cp = pltpu.async_copy(hp_hbm.at[idx_v.at[0]], rows[0], sems[0])
            for k in range(_SUP):
                nxt = None
                if k + 1 < _SUP:
                    nxt = pltpu.async_copy(hp_hbm.at[idx_v.at[k + 1]],
                                           rows[(k + 1) % 2],
                                           sems[(k + 1) % 2])
                cp.wait()
                pltpu.sync_copy(rows[k % 2], acc_sh.at[didx_v.at[k]],
                                add=True)
                cp = nxt
            return carry

        with jax.named_scope("edge_main"):
            lax.fori_loop(0, nsup, sup, 0)

        plsc.subcore_barrier()
        with jax.named_scope("acc_copy_out"):
            pltpu.sync_copy(
                acc_sh.at[pl.ds(s * _RPT, _RPT)],
                out_hbm.at[pl.ds(c * _NPAD + s * _RPT, _RPT)],
            )

    return edge_kernel(hp, src_pad, dst_pad)


def _tc_prescale(x, wt, deg_hists):
    def body(x_ref, wt_ref, dh_ref, hp_ref, dinv_ref):
        hists = dh_ref[...]
        ones_w = jnp.ones((_NW, 1), jnp.float32)
        deg = lax.dot_general(hists, ones_w, (((0,), (0,)), ((), ())),
                              preferred_element_type=jnp.float32,
                  precision=lax.Precision.HIGHEST) + 1.0
        dinv = lax.rsqrt(jnp.maximum(deg, 1e-12))
        dinv_ref[...] = jnp.broadcast_to(dinv, (_NPAD, _DEGW))
        h = jnp.dot(x_ref[...], wt_ref[...], preferred_element_type=jnp.float32,
                  precision=lax.Precision.HIGHEST)
        hp_ref[...] = h * dinv[:_N]

    return pl.pallas_call(
        body,
        out_shape=(
            jax.ShapeDtypeStruct((_N, _D), jnp.float32),
            jax.ShapeDtypeStruct((_NPAD, _DEGW), jnp.float32),
        ),
    )(x, wt, deg_hists)


def _tc_tail(accp, hp, dinv16, batch2, bg, wa_t, ba, qwf,
             xm, mm, cm, signs_t, wc1, wc2, bc):
    def body(accp_ref, hp_ref, dinv_ref, batch_ref, bg_ref, wa_ref, ba_ref,
             qwf_ref, xm_ref, mm_ref, cm_ref, st_ref, wc1_ref, wc2_ref,
             bc_ref, out_ref):
        av = accp_ref[...]
        a = av[:_N] + av[_NPAD:_NPAD + _N]
        dinv = dinv_ref[...][:_N, 0:1]
        g = dinv * (a + hp_ref[...]) + bg_ref[...]
        hr = jnp.maximum(g, 0.0)

        bv = batch_ref[...]
        gi = lax.broadcasted_iota(jnp.int32, (_NG, _N), 0)
        msk = (gi == bv).astype(jnp.float32)
        sums = jnp.dot(msk, hr, preferred_element_type=jnp.float32,
                  precision=lax.Precision.HIGHEST)
        counts = jnp.sum(msk, axis=1, keepdims=True)
        pooled = sums / jnp.maximum(counts, 1.0)

        thetas = (jnp.dot(pooled, wa_ref[...], preferred_element_type=jnp.float32,
                  precision=lax.Precision.HIGHEST)
                  + ba_ref[...] + qwf_ref[...])
        col = lax.broadcasted_iota(jnp.int32, (_NG, _DIM), 1)
        state = jnp.where(col == 0, 1.0, 0.0).astype(jnp.float32)
        mmv = mm_ref[...]
        for l in range(_QL):
            for q in range(_NQ):
                th = thetas[:, l * _NQ + q:l * _NQ + q + 1] * 0.5
                cth = jnp.cos(th)
                sth = jnp.sin(th)
                partner = jnp.dot(state, xm_ref[q],
                                  preferred_element_type=jnp.float32,
                  precision=lax.Precision.HIGHEST)
                state = cth * state + sth * (mmv[q:q + 1] * partner)
            state = jnp.dot(state, cm_ref[...],
                            preferred_element_type=jnp.float32,
                  precision=lax.Precision.HIGHEST)

        qout = jnp.dot(state * state, st_ref[...],
                       preferred_element_type=jnp.float32,
                  precision=lax.Precision.HIGHEST)
        out_ref[...] = (jnp.dot(pooled, wc1_ref[...],
                                preferred_element_type=jnp.float32,
                  precision=lax.Precision.HIGHEST)
                        + jnp.dot(qout, wc2_ref[...],
                                  preferred_element_type=jnp.float32,
                  precision=lax.Precision.HIGHEST)
                        + bc_ref[...])

    return pl.pallas_call(
        body,
        out_shape=jax.ShapeDtypeStruct((_NG, _NCLS), jnp.float32),
    )(accp, hp, dinv16, batch2, bg, wa_t, ba, qwf, xm, mm, cm, signs_t,
      wc1, wc2, bc)


def kernel(x, edge_index, batch, W_gcn, b_gcn, W_ang, b_ang, q_weights,
           W_cls, b_cls):
    src = edge_index[0].astype(jnp.int32)
    dst = edge_index[1].astype(jnp.int32)
    pad = _EPAD - _E
    pad_iota = jnp.arange(pad, dtype=jnp.int32)
    src_pad = jnp.concatenate(
        [src, pad_iota * 37 % _N]).reshape(_NW * _CPT, _CHUNK)
    pad_dst = _N + (pad_iota % (_NPAD - _N))
    dst_pad = jnp.concatenate([dst, pad_dst]).reshape(_NW * _CPT, _CHUNK)

    deg_hists = _sc_degree(dst_pad).reshape(_NW, _NPAD)
    hp, dinv16 = _tc_prescale(x, W_gcn.T, deg_hists)
    accp = _sc_edge_scatter(hp, src_pad, dst_pad)

    out = _tc_tail(
        accp, hp, dinv16,
        batch.astype(jnp.int32).reshape(1, _N),
        b_gcn.reshape(1, _D), W_ang.T, b_ang.reshape(1, -1),
        q_weights.reshape(1, -1),
        jnp.asarray(_XM), jnp.asarray(_MM), jnp.asarray(_CCM),
        jnp.asarray(_SIGNS_T),
        W_cls[:, :_D].T, W_cls[:, _D:].T, b_cls.reshape(1, -1),
    )
    return out
